# baseline (speedup 1.0000x reference)
import jax
import jax.numpy as jnp
from jax import lax
from jax.experimental import pallas as pl
from jax.experimental.pallas import tpu as pltpu
from jax.experimental.pallas import tpu_sc as plsc

N = 10000
D = 256
E = 160000
DH = 128
NC = 2
NS = 16
NW = NC * NS
KB = 128
NB0 = 16
NB1 = 64
NBT = NS * (NB0 + NB1)
EPAD = NBT * KB
NBC = NBT // NW
NP = 10112
RPT = NP // NS
BLK = 1000
CW = 128
F32 = jnp.float32


def _agg_one_core(NB, base0, c, s, hnA, hnB, edges, zrows, PA, PB,
                  src2d, dst2d, dst1, rows0, rows1, acc, sem0, sem1):
    base = base0 + s * NB
    pltpu.sync_copy(edges.at[0, pl.ds(base, NB)], src2d.at[pl.ds(0, NB)])
    pltpu.sync_copy(edges.at[1, pl.ds(base, NB)], dst2d.at[pl.ds(0, NB)])
    bufs = (rows0, rows1)
    sems = (sem0, sem1)

    for half in range(2):
        tab = hnA if half == 0 else hnB
        Pout = PA if half == 0 else PB

        pltpu.sync_copy(zrows.at[pl.ds(s * RPT, RPT)],
                        acc.at[pl.ds(s * RPT, RPT)])
        plsc.subcore_barrier()

        pltpu.async_copy(tab.at[src2d.at[0]], rows0, sem0)
        pltpu.async_copy(tab.at[src2d.at[1]], rows1, sem1)

        @pl.loop(0, NB, step=2)
        def _blk(j):
            for b in range(2):
                jj = j + b
                pltpu.make_async_copy(tab.at[src2d.at[jj]],
                                      bufs[b], sems[b]).wait()
                for k in range(KB // 16):
                    dst1[pl.ds(k * 16, 16)] = dst2d[jj, pl.ds(k * 16, 16)]
                pltpu.sync_copy(bufs[b], acc.at[dst1], add=True)

                @pl.when(jj + 2 < NB)
                def _():
                    pltpu.async_copy(tab.at[src2d.at[jj + 2]],
                                     bufs[b], sems[b])

        plsc.subcore_barrier()
        pltpu.sync_copy(acc.at[pl.ds(s * RPT, RPT)],
                        Pout.at[c, pl.ds(s * RPT, RPT)])


def _sc_agg_body(hnA, hnB, edges, zrows, PA, PB,
                 src2d, dst2d, dst1, rows0, rows1, acc, sem0, sem1):
    c = lax.axis_index("c")
    s = lax.axis_index("s")
    for cc, NB, base0 in ((0, NB0, 0), (1, NB1, NS * NB0)):
        @pl.when(c == cc)
        def _(NB=NB, base0=base0):
            _agg_one_core(NB, base0, c, s, hnA, hnB, edges, zrows, PA, PB,
                          src2d, dst2d, dst1, rows0, rows1, acc, sem0, sem1)


_sc_agg = pl.kernel(
    _sc_agg_body,
    out_type=[
        jax.ShapeDtypeStruct((NC, NP, DH), F32),
        jax.ShapeDtypeStruct((NC, NP, DH), F32),
    ],
    mesh=plsc.VectorSubcoreMesh(core_axis_name="c", subcore_axis_name="s"),
    scratch_types=[
        pltpu.VMEM((NB1, KB), jnp.int32),
        pltpu.VMEM((NB1, KB), jnp.int32),
        pltpu.VMEM((KB,), jnp.int32),
        pltpu.VMEM((KB, DH), F32),
        pltpu.VMEM((KB, DH), F32),
        pltpu.VMEM_SHARED((NP, DH), F32),
        pltpu.SemaphoreType.DMA,
        pltpu.SemaphoreType.DMA,
    ],
)


def _sc_counts_body(edges, zcnt, ones1, C, dst2d, dst1, onesv, cnt):
    c = lax.axis_index("c")
    s = lax.axis_index("s")
    base = (s * NC + c) * NBC

    pltpu.sync_copy(edges.at[1, pl.ds(base, NBC)], dst2d)
    pltpu.sync_copy(ones1, onesv)
    pltpu.sync_copy(zcnt.at[pl.ds(s * RPT, RPT)],
                    cnt.at[pl.ds(s * RPT, RPT)])
    plsc.subcore_barrier()

    @pl.loop(0, NBC)
    def _blk(j):
        for k in range(KB // 16):
            dst1[pl.ds(k * 16, 16)] = dst2d[j, pl.ds(k * 16, 16)]
        pltpu.sync_copy(onesv, cnt.at[dst1], add=True)

    plsc.subcore_barrier()
    pltpu.sync_copy(cnt.at[pl.ds(s * RPT, RPT)],
                    C.at[c, pl.ds(s * RPT, RPT)])


_sc_counts = pl.kernel(
    _sc_counts_body,
    out_type=[jax.ShapeDtypeStruct((NC, NP, CW), F32)],
    mesh=plsc.VectorSubcoreMesh(core_axis_name="c", subcore_axis_name="s"),
    scratch_types=[
        pltpu.VMEM((NBC, KB), jnp.int32),
        pltpu.VMEM((KB,), jnp.int32),
        pltpu.VMEM((KB, CW), F32),
        pltpu.VMEM_SHARED((NP, CW), F32),
    ],
)


def _dot(a, b):
    return jnp.dot(a, b, preferred_element_type=F32,
                   precision=lax.Precision.HIGHEST)


def _normalize(h):
    n = jnp.sqrt(jnp.sum(h * h, axis=1, keepdims=True))
    return h / jnp.maximum(n, 1e-12)


def _silu(z):
    return z * jax.nn.sigmoid(z)


def _norm_split_body(x_ref, a_ref, b_ref):
    hn = _normalize(x_ref[...])
    a_ref[...] = hn[:, :DH]
    b_ref[...] = hn[:, DH:]


def _sage_update(hin_ref, hnA_ref, hnB_ref, PA_ref, PB_ref, C_ref,
                 Wl_ref, bl_ref, Wr_ref):
    inv = 1.0 / jnp.maximum(C_ref[0, :, :1] + C_ref[1, :, :1], 1.0)
    agg = jnp.concatenate([PA_ref[0] + PA_ref[1],
                           PB_ref[0] + PB_ref[1]], axis=1) * inv
    hn = jnp.concatenate([hnA_ref[...], hnB_ref[...]], axis=1)
    z = _dot(agg, Wl_ref[...]) + bl_ref[...] + _dot(hn, Wr_ref[...])
    return _silu(z) + hin_ref[...]


def _layer_body(hin_ref, hnA_ref, hnB_ref, PA_ref, PB_ref, C_ref,
                Wl_ref, bl_ref, Wr_ref, hout_ref, oA_ref, oB_ref):
    h = _sage_update(hin_ref, hnA_ref, hnB_ref, PA_ref, PB_ref, C_ref,
                     Wl_ref, bl_ref, Wr_ref)
    hout_ref[...] = h
    hn2 = _normalize(h)
    oA_ref[...] = hn2[:, :DH]
    oB_ref[...] = hn2[:, DH:]


def _final_body(hin_ref, hnA_ref, hnB_ref, PA_ref, PB_ref, C_ref,
                Wl_ref, bl_ref, Wr_ref, Wfc_ref, bfc_ref, out_ref):
    h = _sage_update(hin_ref, hnA_ref, hnB_ref, PA_ref, PB_ref, C_ref,
                     Wl_ref, bl_ref, Wr_ref)
    hn2 = _normalize(h)
    g = _silu(_dot(hn2, Wfc_ref[...]) + bfc_ref[...]) + h
    out_ref[...] = _normalize(g)


_row = lambda i: (i, 0)
_part = lambda i: (0, i, 0)
_full2 = lambda i: (0, 0)

_SPEC_HD = pl.BlockSpec((BLK, D), _row)
_SPEC_HH = pl.BlockSpec((BLK, DH), _row)
_SPEC_P = pl.BlockSpec((NC, BLK, DH), _part)
_SPEC_C = pl.BlockSpec((NC, BLK, CW), _part)
_SPEC_W = pl.BlockSpec((D, D), _full2)
_SPEC_B = pl.BlockSpec((1, D), _full2)

_GRID = (N // BLK,)

_norm_split = pl.pallas_call(
    _norm_split_body,
    grid=_GRID,
    in_specs=[_SPEC_HD],
    out_specs=[_SPEC_HH, _SPEC_HH],
    out_shape=[jax.ShapeDtypeStruct((NP, DH), F32)] * 2,
)

_layer = pl.pallas_call(
    _layer_body,
    grid=_GRID,
    in_specs=[_SPEC_HD, _SPEC_HH, _SPEC_HH, _SPEC_P, _SPEC_P, _SPEC_C,
              _SPEC_W, _SPEC_B, _SPEC_W],
    out_specs=[_SPEC_HD, _SPEC_HH, _SPEC_HH],
    out_shape=[jax.ShapeDtypeStruct((N, D), F32),
               jax.ShapeDtypeStruct((NP, DH), F32),
               jax.ShapeDtypeStruct((NP, DH), F32)],
)

_final = pl.pallas_call(
    _final_body,
    grid=_GRID,
    in_specs=[_SPEC_HD, _SPEC_HH, _SPEC_HH, _SPEC_P, _SPEC_P, _SPEC_C,
              _SPEC_W, _SPEC_B, _SPEC_W, _SPEC_W, _SPEC_B],
    out_specs=_SPEC_HD,
    out_shape=jax.ShapeDtypeStruct((N, D), F32),
)


def kernel(x, edge_index, W1l, b1l, W1r, W2l, b2l, W2r, Wfc, bfc):
    e = edge_index.astype(jnp.int32)
    pad = jnp.stack([jnp.full((EPAD - E,), N, jnp.int32),
                     jnp.full((EPAD - E,), NP - 1, jnp.int32)])
    edges = jnp.concatenate([e, pad], axis=1).reshape(2, NBT, KB)
    zrows = jnp.zeros((NP, DH), F32)
    ones1 = jnp.ones((KB, CW), F32)

    hnA, hnB = _norm_split(x)
    (C,) = _sc_counts(edges, zrows, ones1)
    PA1, PB1 = _sc_agg(hnA, hnB, edges, zrows)
    h1, hnA2, hnB2 = _layer(x, hnA, hnB, PA1, PB1, C,
                            W1l, b1l.reshape(1, D), W1r)
    PA2, PB2 = _sc_agg(hnA2, hnB2, edges, zrows)
    return _final(h1, hnA2, hnB2, PA2, PB2, C,
                  W2l, b2l.reshape(1, D), W2r, Wfc, bfc.reshape(1, D))


# speedup vs baseline: 1.0901x; 1.0901x over previous
import jax
import jax.numpy as jnp
from jax import lax
from jax.experimental import pallas as pl
from jax.experimental.pallas import tpu as pltpu
from jax.experimental.pallas import tpu_sc as plsc

N = 10000
D = 256
E = 160000
DH = 128
NC = 2
NS = 16
NW = NC * NS
KB = 128
NBT = 1280
EPAD = NBT * KB
NBC = NBT // NW
KS = 64
NSUB = 2 * NBC
NP = 10112
RPT = NP // NS
BLK = 1000
CW = 128
F32 = jnp.float32


def _sc_agg_body(hnA, hnB, edges, zrows, PA, PB,
                 src2d, dst2d, dst1, r0, r1, r2, r3, acc,
                 g0, g1, g2, g3):
    c = lax.axis_index("c")
    s = lax.axis_index("s")
    base = (s * NC + c) * NBC
    pltpu.sync_copy(edges.at[0, pl.ds(base, NBC)], src2d)
    pltpu.sync_copy(edges.at[1, pl.ds(base, NBC)], dst2d)
    bufs = (r0, r1, r2, r3)
    sems = (g0, g1, g2, g3)

    for half in range(2):
        tab = hnA if half == 0 else hnB
        Pout = PA if half == 0 else PB

        pltpu.sync_copy(zrows.at[pl.ds(s * RPT, RPT)],
                        acc.at[pl.ds(s * RPT, RPT)])
        plsc.subcore_barrier()

        for b in range(4):
            pltpu.async_copy(
                tab.at[src2d.at[b // 2, pl.ds((b % 2) * KS, KS)]],
                bufs[b], sems[b])

        @pl.loop(0, NSUB, step=4)
        def _sub(j):
            for b in range(4):
                t = j + b
                blk = t >> 1
                off = (t & 1) * KS
                pltpu.make_async_copy(
                    tab.at[src2d.at[blk, pl.ds(off, KS)]],
                    bufs[b], sems[b]).wait()
                for k in range(KS // 16):
                    dst1[pl.ds(k * 16, 16)] = dst2d[blk,
                                                    pl.ds(off + k * 16, 16)]
                pltpu.sync_copy(bufs[b], acc.at[dst1], add=True)

                @pl.when(t + 4 < NSUB)
                def _():
                    t4 = t + 4
                    pltpu.async_copy(
                        tab.at[src2d.at[t4 >> 1, pl.ds((t4 & 1) * KS, KS)]],
                        bufs[b], sems[b])

        plsc.subcore_barrier()
        pltpu.sync_copy(acc.at[pl.ds(s * RPT, RPT)],
                        Pout.at[c, pl.ds(s * RPT, RPT)])


_sc_agg = pl.kernel(
    _sc_agg_body,
    out_type=[
        jax.ShapeDtypeStruct((NC, NP, DH), F32),
        jax.ShapeDtypeStruct((NC, NP, DH), F32),
    ],
    mesh=plsc.VectorSubcoreMesh(core_axis_name="c", subcore_axis_name="s"),
    scratch_types=[
        pltpu.VMEM((NBC, KB), jnp.int32),
        pltpu.VMEM((NBC, KB), jnp.int32),
        pltpu.VMEM((KS,), jnp.int32),
        pltpu.VMEM((KS, DH), F32),
        pltpu.VMEM((KS, DH), F32),
        pltpu.VMEM((KS, DH), F32),
        pltpu.VMEM((KS, DH), F32),
        pltpu.VMEM_SHARED((NP, DH), F32),
        pltpu.SemaphoreType.DMA,
        pltpu.SemaphoreType.DMA,
        pltpu.SemaphoreType.DMA,
        pltpu.SemaphoreType.DMA,
    ],
)


def _sc_counts_body(edges, zcnt, ones1, C, dst2d, dst1, onesv, cnt):
    c = lax.axis_index("c")
    s = lax.axis_index("s")
    base = (s * NC + c) * NBC

    pltpu.sync_copy(edges.at[1, pl.ds(base, NBC)], dst2d)
    pltpu.sync_copy(ones1, onesv)
    pltpu.sync_copy(zcnt.at[pl.ds(s * RPT, RPT)],
                    cnt.at[pl.ds(s * RPT, RPT)])
    plsc.subcore_barrier()

    @pl.loop(0, NBC)
    def _blk(j):
        for k in range(KB // 16):
            dst1[pl.ds(k * 16, 16)] = dst2d[j, pl.ds(k * 16, 16)]
        pltpu.sync_copy(onesv, cnt.at[dst1], add=True)

    plsc.subcore_barrier()
    pltpu.sync_copy(cnt.at[pl.ds(s * RPT, RPT)],
                    C.at[c, pl.ds(s * RPT, RPT)])


_sc_counts = pl.kernel(
    _sc_counts_body,
    out_type=[jax.ShapeDtypeStruct((NC, NP, CW), F32)],
    mesh=plsc.VectorSubcoreMesh(core_axis_name="c", subcore_axis_name="s"),
    scratch_types=[
        pltpu.VMEM((NBC, KB), jnp.int32),
        pltpu.VMEM((KB,), jnp.int32),
        pltpu.VMEM((KB, CW), F32),
        pltpu.VMEM_SHARED((NP, CW), F32),
    ],
)


def _dot(a, b):
    return jnp.dot(a, b, preferred_element_type=F32,
                   precision=lax.Precision.HIGHEST)


def _normalize(h):
    n = jnp.sqrt(jnp.sum(h * h, axis=1, keepdims=True))
    return h / jnp.maximum(n, 1e-12)


def _silu(z):
    return z * jax.nn.sigmoid(z)


def _norm_split_body(x_ref, a_ref, b_ref):
    hn = _normalize(x_ref[...])
    a_ref[...] = hn[:, :DH]
    b_ref[...] = hn[:, DH:]


def _sage_update(hin_ref, hnA_ref, hnB_ref, PA_ref, PB_ref, C_ref,
                 Wl_ref, bl_ref, Wr_ref):
    inv = 1.0 / jnp.maximum(C_ref[0, :, :1] + C_ref[1, :, :1], 1.0)
    agg = jnp.concatenate([PA_ref[0] + PA_ref[1],
                           PB_ref[0] + PB_ref[1]], axis=1) * inv
    hn = jnp.concatenate([hnA_ref[...], hnB_ref[...]], axis=1)
    z = _dot(agg, Wl_ref[...]) + bl_ref[...] + _dot(hn, Wr_ref[...])
    return _silu(z) + hin_ref[...]


def _layer_body(hin_ref, hnA_ref, hnB_ref, PA_ref, PB_ref, C_ref,
                Wl_ref, bl_ref, Wr_ref, hout_ref, oA_ref, oB_ref):
    h = _sage_update(hin_ref, hnA_ref, hnB_ref, PA_ref, PB_ref, C_ref,
                     Wl_ref, bl_ref, Wr_ref)
    hout_ref[...] = h
    hn2 = _normalize(h)
    oA_ref[...] = hn2[:, :DH]
    oB_ref[...] = hn2[:, DH:]


def _final_body(hin_ref, hnA_ref, hnB_ref, PA_ref, PB_ref, C_ref,
                Wl_ref, bl_ref, Wr_ref, Wfc_ref, bfc_ref, out_ref):
    h = _sage_update(hin_ref, hnA_ref, hnB_ref, PA_ref, PB_ref, C_ref,
                     Wl_ref, bl_ref, Wr_ref)
    hn2 = _normalize(h)
    g = _silu(_dot(hn2, Wfc_ref[...]) + bfc_ref[...]) + h
    out_ref[...] = _normalize(g)


_row = lambda i: (i, 0)
_part = lambda i: (0, i, 0)
_full2 = lambda i: (0, 0)

_SPEC_HD = pl.BlockSpec((BLK, D), _row)
_SPEC_HH = pl.BlockSpec((BLK, DH), _row)
_SPEC_P = pl.BlockSpec((NC, BLK, DH), _part)
_SPEC_C = pl.BlockSpec((NC, BLK, CW), _part)
_SPEC_W = pl.BlockSpec((D, D), _full2)
_SPEC_B = pl.BlockSpec((1, D), _full2)

_GRID = (N // BLK,)

_norm_split = pl.pallas_call(
    _norm_split_body,
    grid=_GRID,
    in_specs=[_SPEC_HD],
    out_specs=[_SPEC_HH, _SPEC_HH],
    out_shape=[jax.ShapeDtypeStruct((NP, DH), F32)] * 2,
)

_layer = pl.pallas_call(
    _layer_body,
    grid=_GRID,
    in_specs=[_SPEC_HD, _SPEC_HH, _SPEC_HH, _SPEC_P, _SPEC_P, _SPEC_C,
              _SPEC_W, _SPEC_B, _SPEC_W],
    out_specs=[_SPEC_HD, _SPEC_HH, _SPEC_HH],
    out_shape=[jax.ShapeDtypeStruct((N, D), F32),
               jax.ShapeDtypeStruct((NP, DH), F32),
               jax.ShapeDtypeStruct((NP, DH), F32)],
)

_final = pl.pallas_call(
    _final_body,
    grid=_GRID,
    in_specs=[_SPEC_HD, _SPEC_HH, _SPEC_HH, _SPEC_P, _SPEC_P, _SPEC_C,
              _SPEC_W, _SPEC_B, _SPEC_W, _SPEC_W, _SPEC_B],
    out_specs=_SPEC_HD,
    out_shape=jax.ShapeDtypeStruct((N, D), F32),
)


def kernel(x, edge_index, W1l, b1l, W1r, W2l, b2l, W2r, Wfc, bfc):
    e = edge_index.astype(jnp.int32)
    pad = jnp.stack([jnp.full((EPAD - E,), N, jnp.int32),
                     jnp.full((EPAD - E,), NP - 1, jnp.int32)])
    edges = jnp.concatenate([e, pad], axis=1).reshape(2, NBT, KB)
    zrows = jnp.zeros((NP, DH), F32)
    ones1 = jnp.ones((KB, CW), F32)

    hnA, hnB = _norm_split(x)
    (C,) = _sc_counts(edges, zrows, ones1)
    PA1, PB1 = _sc_agg(hnA, hnB, edges, zrows)
    h1, hnA2, hnB2 = _layer(x, hnA, hnB, PA1, PB1, C,
                            W1l, b1l.reshape(1, D), W1r)
    PA2, PB2 = _sc_agg(hnA2, hnB2, edges, zrows)
    return _final(h1, hnA2, hnB2, PA2, PB2, C,
                  W2l, b2l.reshape(1, D), W2r, Wfc, bfc.reshape(1, D))
